# baseline (speedup 1.0000x reference)
import functools

import jax
import jax.numpy as jnp
from jax import lax
from jax.experimental import pallas as pl
from jax.experimental.pallas import tpu as pltpu
from jax.experimental.pallas import tpu_sc as plsc

NUM_CAT = 26
OUT_DIMS = 16
MOST_CAT = 100000
MIN_SIG = 1e-4

NW = 32
CHUNK = 1024
GSIZE = 128
NG = CHUNK // GSIZE

_C = (
    3.5075520537e-06 + MIN_SIG,
    0.99979243573,
    -0.49697791117,
    0.31459053537,
    -0.18878267362,
    0.081726808375,
    -0.017208061121,
)


@functools.lru_cache(maxsize=None)
def _build(batch: int):
    n = batch * NUM_CAT
    per_w = n // NW
    nchunk = per_w // CHUNK
    assert per_w % CHUNK == 0

    def body(x_hbm, mask_hbm, mu_hbm, ps_hbm, mu_out, sig_out,
             x_v, m_v, idx_v, mu_v, ps_v, sem):
        wid = lax.axis_index("s") * 2 + lax.axis_index("c")
        base = wid * per_w

        def chunk_body(ci, carry):
            fb = base + ci * CHUNK
            pltpu.sync_copy(x_hbm.at[pl.ds(fb, CHUNK)], x_v)
            pltpu.sync_copy(mask_hbm.at[pl.ds(fb, CHUNK)], m_v)

            def idx_body(k, c2):
                off = k * 16
                g = lax.iota(jnp.int32, 16) + (fb + off)
                shift = lax.rem(g, NUM_CAT) * (MOST_CAT + 1)
                xv = x_v[pl.ds(off, 16)]
                mv = m_v[pl.ds(off, 16)]
                idx_v[pl.ds(off, 16)] = ((xv + 1.0) * mv).astype(jnp.int32) + shift
                return c2

            lax.fori_loop(0, CHUNK // 16, idx_body, 0)

            copies = []
            for j in range(NG):
                isl = idx_v.at[pl.ds(j * GSIZE, GSIZE)]
                dsl = pl.ds(j * GSIZE, GSIZE)
                copies.append(pltpu.async_copy(mu_hbm.at[isl], mu_v.at[dsl], sem))
                copies.append(pltpu.async_copy(ps_hbm.at[isl], ps_v.at[dsl], sem))
            for cp in copies:
                cp.wait()

            pltpu.sync_copy(mu_v, mu_out.at[pl.ds(fb, CHUNK)])

            def sp_body(r, c2):
                v = ps_v[r, :]
                e = jnp.exp(-jnp.abs(v))
                p = jnp.float32(_C[6])
                for coef in (_C[5], _C[4], _C[3], _C[2], _C[1], _C[0]):
                    p = p * e + jnp.float32(coef)
                ps_v[r, :] = jnp.maximum(v, 0.0) + p
                return c2

            lax.fori_loop(0, CHUNK, sp_body, 0)

            pltpu.sync_copy(ps_v, sig_out.at[pl.ds(fb, CHUNK)])
            return carry

        lax.fori_loop(0, nchunk, chunk_body, 0)

    return pl.kernel(
        body,
        out_type=(
            jax.ShapeDtypeStruct((n, OUT_DIMS), jnp.float32),
            jax.ShapeDtypeStruct((n, OUT_DIMS), jnp.float32),
        ),
        mesh=plsc.VectorSubcoreMesh(core_axis_name="c", subcore_axis_name="s"),
        compiler_params=pltpu.CompilerParams(use_tc_tiling_on_sc=False),
        scratch_types=[
            pltpu.VMEM((CHUNK,), jnp.float32),
            pltpu.VMEM((CHUNK,), jnp.float32),
            pltpu.VMEM((CHUNK,), jnp.int32),
            pltpu.VMEM((CHUNK, OUT_DIMS), jnp.float32),
            pltpu.VMEM((CHUNK, OUT_DIMS), jnp.float32),
            pltpu.SemaphoreType.DMA,
        ],
    )


def kernel(x, mask, mu_embeddings, presig_embeddings):
    batch = x.shape[0]
    mu_flat, sig_flat = _build(batch)(
        x.reshape(-1), mask.reshape(-1), mu_embeddings, presig_embeddings)
    return (mu_flat.reshape(batch, NUM_CAT * OUT_DIMS),
            sig_flat.reshape(batch, NUM_CAT * OUT_DIMS))


# speedup vs baseline: 1.0412x; 1.0412x over previous
import functools

import jax
import jax.numpy as jnp
from jax import lax
from jax.experimental import pallas as pl
from jax.experimental.pallas import tpu as pltpu
from jax.experimental.pallas import tpu_sc as plsc

NUM_CAT = 26
OUT_DIMS = 16
MOST_CAT = 100000
MIN_SIG = 1e-4

NW = 32
CHUNK = 1024
GSIZE = 128
NG = CHUNK // GSIZE

_C = (
    3.5075520537e-06 + MIN_SIG,
    0.99979243573,
    -0.49697791117,
    0.31459053537,
    -0.18878267362,
    0.081726808375,
    -0.017208061121,
)


@functools.lru_cache(maxsize=None)
def _build(batch: int):
    n = batch * NUM_CAT
    per_w = n // NW
    nchunk = per_w // CHUNK
    assert per_w % CHUNK == 0

    def body(idx_hbm, mu_hbm, ps_hbm, mu_out, sig_out,
             idx_v, mu_v, ps_v, sem):
        wid = lax.axis_index("s") * 2 + lax.axis_index("c")
        base = wid * per_w

        def chunk_body(ci, carry):
            fb = base + ci * CHUNK
            pltpu.sync_copy(idx_hbm.at[pl.ds(fb, CHUNK)], idx_v)

            copies = []
            for j in range(NG):
                isl = idx_v.at[pl.ds(j * GSIZE, GSIZE)]
                dsl = pl.ds(j * GSIZE, GSIZE)
                copies.append(pltpu.async_copy(mu_hbm.at[isl], mu_v.at[dsl], sem))
                copies.append(pltpu.async_copy(ps_hbm.at[isl], ps_v.at[dsl], sem))
            for cp in copies:
                cp.wait()

            pltpu.sync_copy(mu_v, mu_out.at[pl.ds(fb, CHUNK)])

            def sp_body(r, c2):
                v = ps_v[r, :]
                e = jnp.exp(-jnp.abs(v))
                p = jnp.float32(_C[6])
                for coef in (_C[5], _C[4], _C[3], _C[2], _C[1], _C[0]):
                    p = p * e + jnp.float32(coef)
                ps_v[r, :] = jnp.maximum(v, 0.0) + p
                return c2

            lax.fori_loop(0, CHUNK, sp_body, 0, unroll=4)

            pltpu.sync_copy(ps_v, sig_out.at[pl.ds(fb, CHUNK)])
            return carry

        lax.fori_loop(0, nchunk, chunk_body, 0)

    return pl.kernel(
        body,
        out_type=(
            jax.ShapeDtypeStruct((n, OUT_DIMS), jnp.float32),
            jax.ShapeDtypeStruct((n, OUT_DIMS), jnp.float32),
        ),
        mesh=plsc.VectorSubcoreMesh(core_axis_name="c", subcore_axis_name="s"),
        compiler_params=pltpu.CompilerParams(use_tc_tiling_on_sc=False),
        scratch_types=[
            pltpu.VMEM((CHUNK,), jnp.int32),
            pltpu.VMEM((CHUNK, OUT_DIMS), jnp.float32),
            pltpu.VMEM((CHUNK, OUT_DIMS), jnp.float32),
            pltpu.SemaphoreType.DMA,
        ],
    )


def kernel(x, mask, mu_embeddings, presig_embeddings):
    batch = x.shape[0]
    shift = (jnp.arange(NUM_CAT, dtype=jnp.int32) * (MOST_CAT + 1))[None, :]
    idx = ((x + 1.0) * mask).astype(jnp.int32) + shift
    mu_flat, sig_flat = _build(batch)(
        idx.reshape(-1), mu_embeddings, presig_embeddings)
    return (mu_flat.reshape(batch, NUM_CAT * OUT_DIMS),
            sig_flat.reshape(batch, NUM_CAT * OUT_DIMS))
